# baseline (speedup 1.0000x reference)
import math

import jax
import jax.numpy as jnp
from jax import lax
from jax.experimental import pallas as pl
from jax.experimental.pallas import tpu as pltpu
from jax.experimental.pallas import tpu_sc as plsc

NUM_CORES = 2
NUM_SUBCORES = 16
LANES = 16
NW = NUM_CORES * NUM_SUBCORES

BATCH = 16384
HIDDEN = 64
NROWS = 1000000
QROWS = NROWS // 4
QWORDS = 2 * HIDDEN
NWORDS = HIDDEN // 2
B_PER_W = BATCH // NW
CHUNK = 128
NCHUNKS = B_PER_W // CHUNK
RS_BLK = 8192


def _rowsum_body(x_ref, o_ref):
    o_ref[...] = jnp.sum(x_ref[...], axis=0)


def _rowsum(t_cols):
    grid = (math.ceil(NROWS / RS_BLK),)
    return pl.pallas_call(
        _rowsum_body,
        grid=grid,
        in_specs=[pl.BlockSpec((HIDDEN, RS_BLK), lambda i: (0, i))],
        out_specs=pl.BlockSpec((RS_BLK,), lambda i: (i,)),
        out_shape=jax.ShapeDtypeStruct((NROWS,), jnp.float32),
    )(t_cols)


def _sc_body(users_hbm, items_hbm, fu_hbm, fi_hbm, bu_hbm, bi_hbm, out_hbm,
             raw_u, raw_i, qid_u, qid_i, fub, fib, bub, bib, out_buf, sem):
    wid = lax.axis_index("s") * NUM_CORES + lax.axis_index("c")
    base = wid * NCHUNKS

    pltpu.sync_copy(users_hbm.at[pl.ds(base, NCHUNKS)], raw_u)
    pltpu.sync_copy(items_hbm.at[pl.ds(base, NCHUNKS)], raw_i)

    lane = jnp.arange(LANES, dtype=jnp.int32)

    for c in range(NCHUNKS):
        for g in range(CHUNK // LANES):
            s = pl.ds(g * LANES, LANES)
            qid_u.at[c][s] = raw_u.at[c][s] >> 2
            qid_i.at[c][s] = raw_i.at[c][s] >> 2

    for c in range(NCHUNKS):
        cps = [
            pltpu.async_copy(fu_hbm.at[qid_u.at[c]], fub, sem),
            pltpu.async_copy(fi_hbm.at[qid_i.at[c]], fib, sem),
            pltpu.async_copy(bu_hbm.at[raw_u.at[c]], bub, sem),
            pltpu.async_copy(bi_hbm.at[raw_i.at[c]], bib, sem),
        ]
        for cp in cps:
            cp.wait()

        for g in range(CHUNK // LANES):
            s = pl.ds(g * LANES, LANES)
            row = g * LANES + lane
            wu0 = (raw_u.at[c][s] & 3) << 5
            wi0 = (raw_i.at[c][s] & 3) << 5
            acc0 = bub[s] + bib[s]

            def w_step(w, acc):
                uw = plsc.load_gather(fub, [row, wu0 + w])
                vw = plsc.load_gather(fib, [row, wi0 + w])
                ua, ub = plsc.unpack(
                    plsc.bitcast(uw, jnp.bfloat16),
                    format=plsc.PackFormat.INTERLEAVED,
                    preferred_element_type=jnp.float32)
                va, vb = plsc.unpack(
                    plsc.bitcast(vw, jnp.bfloat16),
                    format=plsc.PackFormat.INTERLEAVED,
                    preferred_element_type=jnp.float32)
                return acc + ua * va + ub * vb

            acc = lax.fori_loop(0, NWORDS, w_step, acc0)
            out_buf[pl.ds(c * CHUNK + g * LANES, LANES)] = acc

    pltpu.sync_copy(out_buf, out_hbm.at[pl.ds(wid * B_PER_W, B_PER_W)])


@jax.jit
def _run(users2d, items2d, fu_q, fi_q, bu_sum, bi_sum):
    mesh = plsc.VectorSubcoreMesh(
        core_axis_name="c", subcore_axis_name="s",
        num_cores=NUM_CORES, num_subcores=NUM_SUBCORES)
    return pl.kernel(
        _sc_body,
        out_type=jax.ShapeDtypeStruct((BATCH,), jnp.float32),
        mesh=mesh,
        compiler_params=pltpu.CompilerParams(
            needs_layout_passes=False, use_tc_tiling_on_sc=True),
        scratch_types=[
            pltpu.VMEM((NCHUNKS, CHUNK), jnp.int32),
            pltpu.VMEM((NCHUNKS, CHUNK), jnp.int32),
            pltpu.VMEM((NCHUNKS, CHUNK), jnp.int32),
            pltpu.VMEM((NCHUNKS, CHUNK), jnp.int32),
            pltpu.VMEM((CHUNK, QWORDS), jnp.int32),
            pltpu.VMEM((CHUNK, QWORDS), jnp.int32),
            pltpu.VMEM((CHUNK,), jnp.float32),
            pltpu.VMEM((CHUNK,), jnp.float32),
            pltpu.VMEM((B_PER_W,), jnp.float32),
            pltpu.SemaphoreType.DMA,
        ],
    )(users2d, items2d, fu_q, fi_q, bu_sum, bi_sum)


def _pack_quads(table):
    b = lax.bitcast_convert_type(table.astype(jnp.bfloat16), jnp.uint16)
    w = b[:, 0::2].astype(jnp.uint32) | (b[:, 1::2].astype(jnp.uint32) << 16)
    q = jnp.concatenate([w[0::4], w[1::4], w[2::4], w[3::4]], axis=1)
    return lax.bitcast_convert_type(q, jnp.int32)


def kernel(users, items, user_factors, item_factors, user_biases,
           item_biases):
    grid = (NW * NCHUNKS, CHUNK)
    users2d = users.reshape(grid)
    items2d = items.reshape(grid)
    fu_q = _pack_quads(user_factors)
    fi_q = _pack_quads(item_factors)
    bu_sum = _rowsum(jnp.swapaxes(user_biases, 0, 1))
    bi_sum = _rowsum(jnp.swapaxes(item_biases, 0, 1))
    out = _run(users2d, items2d, fu_q, fi_q, bu_sum, bi_sum)
    return out.reshape(BATCH, 1)


# speedup vs baseline: 16.5991x; 16.5991x over previous
import math

import jax
import jax.numpy as jnp
from jax import lax
from jax.experimental import pallas as pl
from jax.experimental.pallas import tpu as pltpu
from jax.experimental.pallas import tpu_sc as plsc

NUM_CORES = 2
NUM_SUBCORES = 16
LANES = 16
NW = NUM_CORES * NUM_SUBCORES

BATCH = 16384
HIDDEN = 64
NROWS = 1000000
PAIR_BS = 512
PROWS = PAIR_BS * math.ceil(NROWS / (2 * PAIR_BS))
B_PER_W = BATCH // NW
CHUNK = 128
NCHUNKS = B_PER_W // CHUNK
RS_BLK = 8192


def _rowsum_body(x_ref, o_ref):
    o_ref[...] = jnp.sum(x_ref[...], axis=0)


def _rowsum(t_cols):
    grid = (math.ceil(NROWS / RS_BLK),)
    return pl.pallas_call(
        _rowsum_body,
        grid=grid,
        in_specs=[pl.BlockSpec((HIDDEN, RS_BLK), lambda i: (0, i))],
        out_specs=pl.BlockSpec((RS_BLK,), lambda i: (i,)),
        out_shape=jax.ShapeDtypeStruct((NROWS,), jnp.float32),
    )(t_cols)


def _pack_body(x1_ref, x2_ref, o_ref):
    z = jnp.concatenate([x1_ref[...], x2_ref[...]], axis=0)
    o_ref[...] = z.T


def _pack_pairs(t_cols):
    grid = (math.ceil(NROWS / (2 * PAIR_BS)),)
    return pl.pallas_call(
        _pack_body,
        grid=grid,
        in_specs=[pl.BlockSpec((HIDDEN, PAIR_BS), lambda i: (0, 2 * i)),
                  pl.BlockSpec((HIDDEN, PAIR_BS), lambda i: (0, 2 * i + 1))],
        out_specs=pl.BlockSpec((PAIR_BS, 2 * HIDDEN), lambda i: (i, 0)),
        out_shape=jax.ShapeDtypeStruct((PROWS, 2 * HIDDEN), jnp.float32),
    )(t_cols, t_cols)


def _sc_body(users_hbm, items_hbm, fu_hbm, fi_hbm, bu_hbm, bi_hbm, out_hbm,
             raw_u, raw_i, pid_u, pid_i, fub, fib, bub, bib, out_buf, sem):
    wid = lax.axis_index("s") * NUM_CORES + lax.axis_index("c")
    base = wid * NCHUNKS

    pltpu.sync_copy(users_hbm.at[pl.ds(base, NCHUNKS)], raw_u)
    pltpu.sync_copy(items_hbm.at[pl.ds(base, NCHUNKS)], raw_i)

    lane = jnp.arange(LANES, dtype=jnp.int32)

    for c in range(NCHUNKS):
        for g in range(CHUNK // LANES):
            s = pl.ds(g * LANES, LANES)
            u = raw_u.at[c][s]
            v = raw_i.at[c][s]
            pid_u.at[c][s] = ((u >> 10) << 9) | (u & 511)
            pid_i.at[c][s] = ((v >> 10) << 9) | (v & 511)

    for c in range(NCHUNKS):
        cps = [
            pltpu.async_copy(fu_hbm.at[pid_u.at[c]], fub, sem),
            pltpu.async_copy(fi_hbm.at[pid_i.at[c]], fib, sem),
            pltpu.async_copy(bu_hbm.at[raw_u.at[c]], bub, sem),
            pltpu.async_copy(bi_hbm.at[raw_i.at[c]], bib, sem),
        ]
        for cp in cps:
            cp.wait()

        for g in range(CHUNK // LANES):
            s = pl.ds(g * LANES, LANES)
            row = g * LANES + lane
            cu0 = ((raw_u.at[c][s] >> 9) & 1) << 6
            ci0 = ((raw_i.at[c][s] >> 9) & 1) << 6
            acc0 = bub[s] + bib[s]

            def h_step(h, acc):
                uu = plsc.load_gather(fub, [row, cu0 + h])
                vv = plsc.load_gather(fib, [row, ci0 + h])
                return acc + uu * vv

            acc = lax.fori_loop(0, HIDDEN, h_step, acc0)
            out_buf[pl.ds(c * CHUNK + g * LANES, LANES)] = acc

    pltpu.sync_copy(out_buf, out_hbm.at[pl.ds(wid * B_PER_W, B_PER_W)])


@jax.jit
def _run(users2d, items2d, fu_p, fi_p, bu_sum, bi_sum):
    mesh = plsc.VectorSubcoreMesh(
        core_axis_name="c", subcore_axis_name="s",
        num_cores=NUM_CORES, num_subcores=NUM_SUBCORES)
    return pl.kernel(
        _sc_body,
        out_type=jax.ShapeDtypeStruct((BATCH,), jnp.float32),
        mesh=mesh,
        compiler_params=pltpu.CompilerParams(
            needs_layout_passes=False, use_tc_tiling_on_sc=True),
        scratch_types=[
            pltpu.VMEM((NCHUNKS, CHUNK), jnp.int32),
            pltpu.VMEM((NCHUNKS, CHUNK), jnp.int32),
            pltpu.VMEM((NCHUNKS, CHUNK), jnp.int32),
            pltpu.VMEM((NCHUNKS, CHUNK), jnp.int32),
            pltpu.VMEM((CHUNK, 2 * HIDDEN), jnp.float32),
            pltpu.VMEM((CHUNK, 2 * HIDDEN), jnp.float32),
            pltpu.VMEM((CHUNK,), jnp.float32),
            pltpu.VMEM((CHUNK,), jnp.float32),
            pltpu.VMEM((B_PER_W,), jnp.float32),
            pltpu.SemaphoreType.DMA,
        ],
    )(users2d, items2d, fu_p, fi_p, bu_sum, bi_sum)


def kernel(users, items, user_factors, item_factors, user_biases,
           item_biases):
    grid = (NW * NCHUNKS, CHUNK)
    users2d = users.reshape(grid)
    items2d = items.reshape(grid)
    fu_p = _pack_pairs(jnp.swapaxes(user_factors, 0, 1))
    fi_p = _pack_pairs(jnp.swapaxes(item_factors, 0, 1))
    bu_sum = _rowsum(jnp.swapaxes(user_biases, 0, 1))
    bi_sum = _rowsum(jnp.swapaxes(item_biases, 0, 1))
    out = _run(users2d, items2d, fu_p, fi_p, bu_sum, bi_sum)
    return out.reshape(BATCH, 1)


# speedup vs baseline: 34.3550x; 2.0697x over previous
import math

import jax
import jax.numpy as jnp
from jax import lax
from jax.experimental import pallas as pl
from jax.experimental.pallas import tpu as pltpu
from jax.experimental.pallas import tpu_sc as plsc

NUM_CORES = 2
NUM_SUBCORES = 16
LANES = 16
NW = NUM_CORES * NUM_SUBCORES

BATCH = 16384
HIDDEN = 64
NROWS = 1000000
PAIR_BS = 2048
PROWS = PAIR_BS * math.ceil(NROWS / (2 * PAIR_BS))
PAIR_SH = PAIR_BS.bit_length() - 1
PAIR_NBLK = math.ceil(NROWS / PAIR_BS)
B_PER_W = BATCH // NW
CHUNK = 128
NCHUNKS = B_PER_W // CHUNK
RS_BLK = 32768


def _rowsum_body(x_ref, o_ref):
    o_ref[...] = jnp.sum(x_ref[...], axis=0)


def _rowsum(t_cols):
    grid = (math.ceil(NROWS / RS_BLK),)
    return pl.pallas_call(
        _rowsum_body,
        grid=grid,
        in_specs=[pl.BlockSpec((HIDDEN, RS_BLK), lambda i: (0, i))],
        out_specs=pl.BlockSpec((RS_BLK,), lambda i: (i,)),
        out_shape=jax.ShapeDtypeStruct((NROWS,), jnp.float32),
    )(t_cols)


def _pack_body(x1_ref, x2_ref, o_ref):
    z = jnp.concatenate([x1_ref[...], x2_ref[...]], axis=0)
    o_ref[...] = z.T


def _pack_pairs(t_cols):
    grid = (math.ceil(NROWS / (2 * PAIR_BS)),)
    nb = PAIR_NBLK
    return pl.pallas_call(
        _pack_body,
        grid=grid,
        in_specs=[
            pl.BlockSpec((HIDDEN, PAIR_BS),
                         lambda i: (0, jnp.minimum(2 * i, nb - 1))),
            pl.BlockSpec((HIDDEN, PAIR_BS),
                         lambda i: (0, jnp.minimum(2 * i + 1, nb - 1))),
        ],
        out_specs=pl.BlockSpec((PAIR_BS, 2 * HIDDEN), lambda i: (i, 0)),
        out_shape=jax.ShapeDtypeStruct((PROWS, 2 * HIDDEN), jnp.float32),
    )(t_cols, t_cols)


def _sc_body(users_hbm, items_hbm, fu_hbm, fi_hbm, bu_hbm, bi_hbm, out_hbm,
             raw_u, raw_i, pid_u, pid_i, fub, fib, bub, bib, out_buf, sem):
    wid = lax.axis_index("s") * NUM_CORES + lax.axis_index("c")
    base = wid * NCHUNKS

    pltpu.sync_copy(users_hbm.at[pl.ds(base, NCHUNKS)], raw_u)
    pltpu.sync_copy(items_hbm.at[pl.ds(base, NCHUNKS)], raw_i)

    lane = jnp.arange(LANES, dtype=jnp.int32)

    for c in range(NCHUNKS):
        for g in range(CHUNK // LANES):
            s = pl.ds(g * LANES, LANES)
            u = raw_u.at[c][s]
            v = raw_i.at[c][s]
            pid_u.at[c][s] = (
                ((u >> (PAIR_SH + 1)) << PAIR_SH) | (u & (PAIR_BS - 1)))
            pid_i.at[c][s] = (
                ((v >> (PAIR_SH + 1)) << PAIR_SH) | (v & (PAIR_BS - 1)))

    for c in range(NCHUNKS):
        cps = [
            pltpu.async_copy(fu_hbm.at[pid_u.at[c]], fub, sem),
            pltpu.async_copy(fi_hbm.at[pid_i.at[c]], fib, sem),
            pltpu.async_copy(bu_hbm.at[raw_u.at[c]], bub, sem),
            pltpu.async_copy(bi_hbm.at[raw_i.at[c]], bib, sem),
        ]
        for cp in cps:
            cp.wait()

        for g in range(CHUNK // LANES):
            s = pl.ds(g * LANES, LANES)
            row = g * LANES + lane
            cu0 = ((raw_u.at[c][s] >> PAIR_SH) & 1) << 6
            ci0 = ((raw_i.at[c][s] >> PAIR_SH) & 1) << 6
            acc0 = bub[s] + bib[s]

            def h_step(h, acc):
                uu = plsc.load_gather(fub, [row, cu0 + h])
                vv = plsc.load_gather(fib, [row, ci0 + h])
                return acc + uu * vv

            acc = lax.fori_loop(0, HIDDEN, h_step, acc0)
            out_buf[pl.ds(c * CHUNK + g * LANES, LANES)] = acc

    pltpu.sync_copy(out_buf, out_hbm.at[pl.ds(wid * B_PER_W, B_PER_W)])


@jax.jit
def _run(users2d, items2d, fu_p, fi_p, bu_sum, bi_sum):
    mesh = plsc.VectorSubcoreMesh(
        core_axis_name="c", subcore_axis_name="s",
        num_cores=NUM_CORES, num_subcores=NUM_SUBCORES)
    return pl.kernel(
        _sc_body,
        out_type=jax.ShapeDtypeStruct((BATCH,), jnp.float32),
        mesh=mesh,
        compiler_params=pltpu.CompilerParams(
            needs_layout_passes=False, use_tc_tiling_on_sc=True),
        scratch_types=[
            pltpu.VMEM((NCHUNKS, CHUNK), jnp.int32),
            pltpu.VMEM((NCHUNKS, CHUNK), jnp.int32),
            pltpu.VMEM((NCHUNKS, CHUNK), jnp.int32),
            pltpu.VMEM((NCHUNKS, CHUNK), jnp.int32),
            pltpu.VMEM((CHUNK, 2 * HIDDEN), jnp.float32),
            pltpu.VMEM((CHUNK, 2 * HIDDEN), jnp.float32),
            pltpu.VMEM((CHUNK,), jnp.float32),
            pltpu.VMEM((CHUNK,), jnp.float32),
            pltpu.VMEM((B_PER_W,), jnp.float32),
            pltpu.SemaphoreType.DMA,
        ],
    )(users2d, items2d, fu_p, fi_p, bu_sum, bi_sum)


def kernel(users, items, user_factors, item_factors, user_biases,
           item_biases):
    grid = (NW * NCHUNKS, CHUNK)
    users2d = users.reshape(grid)
    items2d = items.reshape(grid)
    fu_p = _pack_pairs(jnp.swapaxes(user_factors, 0, 1))
    fi_p = _pack_pairs(jnp.swapaxes(item_factors, 0, 1))
    bu_sum = _rowsum(jnp.swapaxes(user_biases, 0, 1))
    bi_sum = _rowsum(jnp.swapaxes(item_biases, 0, 1))
    out = _run(users2d, items2d, fu_p, fi_p, bu_sum, bi_sum)
    return out.reshape(BATCH, 1)


# speedup vs baseline: 46.3118x; 1.3480x over previous
import math

import jax
import jax.numpy as jnp
from jax import lax
from jax.experimental import pallas as pl
from jax.experimental.pallas import tpu as pltpu
from jax.experimental.pallas import tpu_sc as plsc

NUM_CORES = 2
NUM_SUBCORES = 16
LANES = 16
NW = NUM_CORES * NUM_SUBCORES

BATCH = 16384
HIDDEN = 64
NROWS = 1000000
PAIR_BS = 2048
PROWS = PAIR_BS * math.ceil(NROWS / (2 * PAIR_BS))
PAIR_SH = PAIR_BS.bit_length() - 1
PAIR_NBLK = math.ceil(NROWS / PAIR_BS)
B_PER_W = BATCH // NW
CHUNK = 128
NCHUNKS = B_PER_W // CHUNK
RS_BLK = 32768


def _prep_body(u1, u2, i1, i2, b1, b2, c1, c2, ou_ref, oi_ref,
               obu_ref, obi_ref):
    ou_ref[...] = jnp.concatenate([u1[...], u2[...]], axis=0).T
    oi_ref[...] = jnp.concatenate([i1[...], i2[...]], axis=0).T
    obu_ref[...] = jnp.concatenate(
        [jnp.sum(b1[...], axis=0), jnp.sum(b2[...], axis=0)])
    obi_ref[...] = jnp.concatenate(
        [jnp.sum(c1[...], axis=0), jnp.sum(c2[...], axis=0)])


def _prep(uf_cols, if_cols, ub_cols, ib_cols):
    grid = (math.ceil(NROWS / (2 * PAIR_BS)),)
    nb = PAIR_NBLK

    def even(i):
        return (0, jnp.minimum(2 * i, nb - 1))

    def odd(i):
        return (0, jnp.minimum(2 * i + 1, nb - 1))

    spec_e = pl.BlockSpec((HIDDEN, PAIR_BS), even)
    spec_o = pl.BlockSpec((HIDDEN, PAIR_BS), odd)
    return pl.pallas_call(
        _prep_body,
        grid=grid,
        in_specs=[spec_e, spec_o] * 4,
        out_specs=(
            pl.BlockSpec((PAIR_BS, 2 * HIDDEN), lambda i: (i, 0)),
            pl.BlockSpec((PAIR_BS, 2 * HIDDEN), lambda i: (i, 0)),
            pl.BlockSpec((2 * PAIR_BS,), lambda i: (i,)),
            pl.BlockSpec((2 * PAIR_BS,), lambda i: (i,)),
        ),
        out_shape=(
            jax.ShapeDtypeStruct((PROWS, 2 * HIDDEN), jnp.float32),
            jax.ShapeDtypeStruct((PROWS, 2 * HIDDEN), jnp.float32),
            jax.ShapeDtypeStruct((NROWS,), jnp.float32),
            jax.ShapeDtypeStruct((NROWS,), jnp.float32),
        ),
    )(uf_cols, uf_cols, if_cols, if_cols, ub_cols, ub_cols,
      ib_cols, ib_cols)


def _sc_body(users_hbm, items_hbm, fu_hbm, fi_hbm, bu_hbm, bi_hbm, out_hbm,
             raw_u, raw_i, pid_u, pid_i, fub, fib, bub, bib, out_buf, sem):
    wid = lax.axis_index("s") * NUM_CORES + lax.axis_index("c")
    base = wid * NCHUNKS

    pltpu.sync_copy(users_hbm.at[pl.ds(base, NCHUNKS)], raw_u)
    pltpu.sync_copy(items_hbm.at[pl.ds(base, NCHUNKS)], raw_i)

    lane = jnp.arange(LANES, dtype=jnp.int32)

    for c in range(NCHUNKS):
        for g in range(CHUNK // LANES):
            s = pl.ds(g * LANES, LANES)
            u = raw_u.at[c][s]
            v = raw_i.at[c][s]
            pid_u.at[c][s] = (
                ((u >> (PAIR_SH + 1)) << PAIR_SH) | (u & (PAIR_BS - 1)))
            pid_i.at[c][s] = (
                ((v >> (PAIR_SH + 1)) << PAIR_SH) | (v & (PAIR_BS - 1)))

    for c in range(NCHUNKS):
        cps = [
            pltpu.async_copy(fu_hbm.at[pid_u.at[c]], fub, sem),
            pltpu.async_copy(fi_hbm.at[pid_i.at[c]], fib, sem),
            pltpu.async_copy(bu_hbm.at[raw_u.at[c]], bub, sem),
            pltpu.async_copy(bi_hbm.at[raw_i.at[c]], bib, sem),
        ]
        for cp in cps:
            cp.wait()

        for g in range(CHUNK // LANES):
            s = pl.ds(g * LANES, LANES)
            row = g * LANES + lane
            cu0 = ((raw_u.at[c][s] >> PAIR_SH) & 1) << 6
            ci0 = ((raw_i.at[c][s] >> PAIR_SH) & 1) << 6
            acc0 = bub[s] + bib[s]

            def h_step(h, acc):
                uu = plsc.load_gather(fub, [row, cu0 + h])
                vv = plsc.load_gather(fib, [row, ci0 + h])
                return acc + uu * vv

            acc = lax.fori_loop(0, HIDDEN, h_step, acc0)
            out_buf[pl.ds(c * CHUNK + g * LANES, LANES)] = acc

    pltpu.sync_copy(out_buf, out_hbm.at[pl.ds(wid * B_PER_W, B_PER_W)])


@jax.jit
def _run(users2d, items2d, fu_p, fi_p, bu_sum, bi_sum):
    mesh = plsc.VectorSubcoreMesh(
        core_axis_name="c", subcore_axis_name="s",
        num_cores=NUM_CORES, num_subcores=NUM_SUBCORES)
    return pl.kernel(
        _sc_body,
        out_type=jax.ShapeDtypeStruct((BATCH,), jnp.float32),
        mesh=mesh,
        compiler_params=pltpu.CompilerParams(
            needs_layout_passes=False, use_tc_tiling_on_sc=True),
        scratch_types=[
            pltpu.VMEM((NCHUNKS, CHUNK), jnp.int32),
            pltpu.VMEM((NCHUNKS, CHUNK), jnp.int32),
            pltpu.VMEM((NCHUNKS, CHUNK), jnp.int32),
            pltpu.VMEM((NCHUNKS, CHUNK), jnp.int32),
            pltpu.VMEM((CHUNK, 2 * HIDDEN), jnp.float32),
            pltpu.VMEM((CHUNK, 2 * HIDDEN), jnp.float32),
            pltpu.VMEM((CHUNK,), jnp.float32),
            pltpu.VMEM((CHUNK,), jnp.float32),
            pltpu.VMEM((B_PER_W,), jnp.float32),
            pltpu.SemaphoreType.DMA,
        ],
    )(users2d, items2d, fu_p, fi_p, bu_sum, bi_sum)


def kernel(users, items, user_factors, item_factors, user_biases,
           item_biases):
    grid = (NW * NCHUNKS, CHUNK)
    users2d = users.reshape(grid)
    items2d = items.reshape(grid)
    fu_p, fi_p, bu_sum, bi_sum = _prep(
        jnp.swapaxes(user_factors, 0, 1),
        jnp.swapaxes(item_factors, 0, 1),
        jnp.swapaxes(user_biases, 0, 1),
        jnp.swapaxes(item_biases, 0, 1))
    out = _run(users2d, items2d, fu_p, fi_p, bu_sum, bi_sum)
    return out.reshape(BATCH, 1)


# speedup vs baseline: 55.2335x; 1.1926x over previous
import math

import jax
import jax.numpy as jnp
from jax import lax
from jax.experimental import pallas as pl
from jax.experimental.pallas import tpu as pltpu
from jax.experimental.pallas import tpu_sc as plsc

NUM_CORES = 2
NUM_SUBCORES = 16
LANES = 16
NW = NUM_CORES * NUM_SUBCORES

BATCH = 16384
HIDDEN = 64
NROWS = 1000000
PAIR_BS = 2048
PROWS = PAIR_BS * math.ceil(NROWS / (4 * PAIR_BS))
PAIR_SH = PAIR_BS.bit_length() - 1
PAIR_NBLK = math.ceil(NROWS / PAIR_BS)
B_PER_W = BATCH // NW
CHUNK = 128
NCHUNKS = B_PER_W // CHUNK
RS_BLK = 32768


def _pack4(x0, x1, x2, x3):
    def two(a, b):
        t = jnp.concatenate([a, b], axis=0).T
        w = lax.bitcast_convert_type(
            t.astype(jnp.bfloat16), jnp.uint16)
        return (w[:, :HIDDEN].astype(jnp.uint32)
                | (w[:, HIDDEN:].astype(jnp.uint32) << 16))
    q = jnp.concatenate([two(x0, x1), two(x2, x3)], axis=1)
    return lax.bitcast_convert_type(q, jnp.int32)


def _prep_body(u0, u1, u2, u3, i0, i1, i2, i3, b0, b1, b2, b3,
               c0, c1, c2, c3, ou_ref, oi_ref, obu_ref, obi_ref):
    ou_ref[...] = _pack4(u0[...], u1[...], u2[...], u3[...])
    oi_ref[...] = _pack4(i0[...], i1[...], i2[...], i3[...])
    obu_ref[...] = jnp.concatenate(
        [jnp.sum(b0[...], axis=0), jnp.sum(b1[...], axis=0),
         jnp.sum(b2[...], axis=0), jnp.sum(b3[...], axis=0)])
    obi_ref[...] = jnp.concatenate(
        [jnp.sum(c0[...], axis=0), jnp.sum(c1[...], axis=0),
         jnp.sum(c2[...], axis=0), jnp.sum(c3[...], axis=0)])


def _prep(uf_cols, if_cols, ub_cols, ib_cols):
    grid = (math.ceil(NROWS / (4 * PAIR_BS)),)
    nb = PAIR_NBLK

    def blk(k):
        return pl.BlockSpec(
            (HIDDEN, PAIR_BS), lambda i, k=k: (0, jnp.minimum(4 * i + k,
                                                              nb - 1)))

    return pl.pallas_call(
        _prep_body,
        grid=grid,
        in_specs=[blk(0), blk(1), blk(2), blk(3)] * 4,
        out_specs=(
            pl.BlockSpec((PAIR_BS, 2 * HIDDEN), lambda i: (i, 0)),
            pl.BlockSpec((PAIR_BS, 2 * HIDDEN), lambda i: (i, 0)),
            pl.BlockSpec((4 * PAIR_BS,), lambda i: (i,)),
            pl.BlockSpec((4 * PAIR_BS,), lambda i: (i,)),
        ),
        out_shape=(
            jax.ShapeDtypeStruct((PROWS, 2 * HIDDEN), jnp.int32),
            jax.ShapeDtypeStruct((PROWS, 2 * HIDDEN), jnp.int32),
            jax.ShapeDtypeStruct((NROWS,), jnp.float32),
            jax.ShapeDtypeStruct((NROWS,), jnp.float32),
        ),
    )(uf_cols, uf_cols, uf_cols, uf_cols,
      if_cols, if_cols, if_cols, if_cols,
      ub_cols, ub_cols, ub_cols, ub_cols,
      ib_cols, ib_cols, ib_cols, ib_cols)


def _sc_body(users_hbm, items_hbm, fu_hbm, fi_hbm, bu_hbm, bi_hbm, out_hbm,
             raw_u, raw_i, pid_u, pid_i, fub, fib, bub, bib, out_buf, sem):
    wid = lax.axis_index("s") * NUM_CORES + lax.axis_index("c")
    base = wid * NCHUNKS

    pltpu.sync_copy(users_hbm.at[pl.ds(base, NCHUNKS)], raw_u)
    pltpu.sync_copy(items_hbm.at[pl.ds(base, NCHUNKS)], raw_i)

    lane = jnp.arange(LANES, dtype=jnp.int32)

    for c in range(NCHUNKS):
        for g in range(CHUNK // LANES):
            s = pl.ds(g * LANES, LANES)
            u = raw_u.at[c][s]
            v = raw_i.at[c][s]
            pid_u.at[c][s] = (
                ((u >> (PAIR_SH + 2)) << PAIR_SH) | (u & (PAIR_BS - 1)))
            pid_i.at[c][s] = (
                ((v >> (PAIR_SH + 2)) << PAIR_SH) | (v & (PAIR_BS - 1)))

    for c in range(NCHUNKS):
        cps = [
            pltpu.async_copy(fu_hbm.at[pid_u.at[c]], fub, sem),
            pltpu.async_copy(fi_hbm.at[pid_i.at[c]], fib, sem),
            pltpu.async_copy(bu_hbm.at[raw_u.at[c]], bub, sem),
            pltpu.async_copy(bi_hbm.at[raw_i.at[c]], bib, sem),
        ]
        for cp in cps:
            cp.wait()

        for g in range(CHUNK // LANES):
            s = pl.ds(g * LANES, LANES)
            row = g * LANES + lane
            ru = raw_u.at[c][s]
            ri = raw_i.at[c][s]
            cu0 = ((ru >> (PAIR_SH + 1)) & 1) << 6
            ci0 = ((ri >> (PAIR_SH + 1)) & 1) << 6
            usel = ((ru >> PAIR_SH) & 1) == 1
            isel = ((ri >> PAIR_SH) & 1) == 1
            acc0 = bub[s] + bib[s]

            def h_step(h, acc):
                uw = plsc.load_gather(fub, [row, cu0 + h])
                vw = plsc.load_gather(fib, [row, ci0 + h])
                ulo, uhi = plsc.unpack(
                    plsc.bitcast(uw, jnp.bfloat16),
                    format=plsc.PackFormat.INTERLEAVED,
                    preferred_element_type=jnp.float32)
                vlo, vhi = plsc.unpack(
                    plsc.bitcast(vw, jnp.bfloat16),
                    format=plsc.PackFormat.INTERLEAVED,
                    preferred_element_type=jnp.float32)
                uu = jnp.where(usel, uhi, ulo)
                vv = jnp.where(isel, vhi, vlo)
                return acc + uu * vv

            acc = lax.fori_loop(0, HIDDEN, h_step, acc0)
            out_buf[pl.ds(c * CHUNK + g * LANES, LANES)] = acc

    pltpu.sync_copy(out_buf, out_hbm.at[pl.ds(wid * B_PER_W, B_PER_W)])


@jax.jit
def _run(users2d, items2d, fu_p, fi_p, bu_sum, bi_sum):
    mesh = plsc.VectorSubcoreMesh(
        core_axis_name="c", subcore_axis_name="s",
        num_cores=NUM_CORES, num_subcores=NUM_SUBCORES)
    return pl.kernel(
        _sc_body,
        out_type=jax.ShapeDtypeStruct((BATCH,), jnp.float32),
        mesh=mesh,
        compiler_params=pltpu.CompilerParams(
            needs_layout_passes=False, use_tc_tiling_on_sc=True),
        scratch_types=[
            pltpu.VMEM((NCHUNKS, CHUNK), jnp.int32),
            pltpu.VMEM((NCHUNKS, CHUNK), jnp.int32),
            pltpu.VMEM((NCHUNKS, CHUNK), jnp.int32),
            pltpu.VMEM((NCHUNKS, CHUNK), jnp.int32),
            pltpu.VMEM((CHUNK, 2 * HIDDEN), jnp.int32),
            pltpu.VMEM((CHUNK, 2 * HIDDEN), jnp.int32),
            pltpu.VMEM((CHUNK,), jnp.float32),
            pltpu.VMEM((CHUNK,), jnp.float32),
            pltpu.VMEM((B_PER_W,), jnp.float32),
            pltpu.SemaphoreType.DMA,
        ],
    )(users2d, items2d, fu_p, fi_p, bu_sum, bi_sum)


def kernel(users, items, user_factors, item_factors, user_biases,
           item_biases):
    grid = (NW * NCHUNKS, CHUNK)
    users2d = users.reshape(grid)
    items2d = items.reshape(grid)
    fu_p, fi_p, bu_sum, bi_sum = _prep(
        jnp.swapaxes(user_factors, 0, 1),
        jnp.swapaxes(item_factors, 0, 1),
        jnp.swapaxes(user_biases, 0, 1),
        jnp.swapaxes(item_biases, 0, 1))
    out = _run(users2d, items2d, fu_p, fi_p, bu_sum, bi_sum)
    return out.reshape(BATCH, 1)


# speedup vs baseline: 57.0469x; 1.0328x over previous
import math

import jax
import jax.numpy as jnp
from jax import lax
from jax.experimental import pallas as pl
from jax.experimental.pallas import tpu as pltpu
from jax.experimental.pallas import tpu_sc as plsc

NUM_CORES = 2
NUM_SUBCORES = 16
LANES = 16
NW = NUM_CORES * NUM_SUBCORES

BATCH = 16384
HIDDEN = 64
NROWS = 1000000
PAIR_BS = 4096
PROWS = PAIR_BS * math.ceil(NROWS / (4 * PAIR_BS))
PAIR_SH = PAIR_BS.bit_length() - 1
PAIR_NBLK = math.ceil(NROWS / PAIR_BS)
B_PER_W = BATCH // NW
CHUNK = 128
NCHUNKS = B_PER_W // CHUNK
RS_BLK = 32768


def _pack4(x0, x1, x2, x3):
    def two(a, b):
        t = jnp.concatenate([a, b], axis=0).T
        w = lax.bitcast_convert_type(
            t.astype(jnp.bfloat16), jnp.uint16)
        return (w[:, :HIDDEN].astype(jnp.uint32)
                | (w[:, HIDDEN:].astype(jnp.uint32) << 16))
    q = jnp.concatenate([two(x0, x1), two(x2, x3)], axis=1)
    return lax.bitcast_convert_type(q, jnp.int32)


def _prep_body(u0, u1, u2, u3, i0, i1, i2, i3, b0, b1, b2, b3,
               c0, c1, c2, c3, ou_ref, oi_ref, obu_ref, obi_ref):
    ou_ref[...] = _pack4(u0[...], u1[...], u2[...], u3[...])
    oi_ref[...] = _pack4(i0[...], i1[...], i2[...], i3[...])
    obu_ref[...] = jnp.concatenate(
        [jnp.sum(b0[...], axis=0), jnp.sum(b1[...], axis=0),
         jnp.sum(b2[...], axis=0), jnp.sum(b3[...], axis=0)])
    obi_ref[...] = jnp.concatenate(
        [jnp.sum(c0[...], axis=0), jnp.sum(c1[...], axis=0),
         jnp.sum(c2[...], axis=0), jnp.sum(c3[...], axis=0)])


def _prep(uf_cols, if_cols, ub_cols, ib_cols):
    grid = (math.ceil(NROWS / (4 * PAIR_BS)),)
    nb = PAIR_NBLK

    def blk(k):
        return pl.BlockSpec(
            (HIDDEN, PAIR_BS), lambda i, k=k: (0, jnp.minimum(4 * i + k,
                                                              nb - 1)))

    return pl.pallas_call(
        _prep_body,
        grid=grid,
        in_specs=[blk(0), blk(1), blk(2), blk(3)] * 4,
        out_specs=(
            pl.BlockSpec((PAIR_BS, 2 * HIDDEN), lambda i: (i, 0)),
            pl.BlockSpec((PAIR_BS, 2 * HIDDEN), lambda i: (i, 0)),
            pl.BlockSpec((4 * PAIR_BS,), lambda i: (i,)),
            pl.BlockSpec((4 * PAIR_BS,), lambda i: (i,)),
        ),
        out_shape=(
            jax.ShapeDtypeStruct((PROWS, 2 * HIDDEN), jnp.int32),
            jax.ShapeDtypeStruct((PROWS, 2 * HIDDEN), jnp.int32),
            jax.ShapeDtypeStruct((NROWS,), jnp.float32),
            jax.ShapeDtypeStruct((NROWS,), jnp.float32),
        ),
    )(uf_cols, uf_cols, uf_cols, uf_cols,
      if_cols, if_cols, if_cols, if_cols,
      ub_cols, ub_cols, ub_cols, ub_cols,
      ib_cols, ib_cols, ib_cols, ib_cols)


def _sc_body(users_hbm, items_hbm, fu_hbm, fi_hbm, bu_hbm, bi_hbm, out_hbm,
             raw_u, raw_i, pid_u, pid_i, fub, fib, bub, bib, out_buf, sem):
    wid = lax.axis_index("s") * NUM_CORES + lax.axis_index("c")
    base = wid * NCHUNKS

    pltpu.sync_copy(users_hbm.at[pl.ds(base, NCHUNKS)], raw_u)
    pltpu.sync_copy(items_hbm.at[pl.ds(base, NCHUNKS)], raw_i)

    lane = jnp.arange(LANES, dtype=jnp.int32)

    for c in range(NCHUNKS):
        for g in range(CHUNK // LANES):
            s = pl.ds(g * LANES, LANES)
            u = raw_u.at[c][s]
            v = raw_i.at[c][s]
            pid_u.at[c][s] = (
                ((u >> (PAIR_SH + 2)) << PAIR_SH) | (u & (PAIR_BS - 1)))
            pid_i.at[c][s] = (
                ((v >> (PAIR_SH + 2)) << PAIR_SH) | (v & (PAIR_BS - 1)))

    for c in range(NCHUNKS):
        cps = [
            pltpu.async_copy(fu_hbm.at[pid_u.at[c]], fub, sem),
            pltpu.async_copy(fi_hbm.at[pid_i.at[c]], fib, sem),
            pltpu.async_copy(bu_hbm.at[raw_u.at[c]], bub, sem),
            pltpu.async_copy(bi_hbm.at[raw_i.at[c]], bib, sem),
        ]
        for cp in cps:
            cp.wait()

        for g in range(CHUNK // LANES):
            s = pl.ds(g * LANES, LANES)
            row = g * LANES + lane
            ru = raw_u.at[c][s]
            ri = raw_i.at[c][s]
            cu0 = ((ru >> (PAIR_SH + 1)) & 1) << 6
            ci0 = ((ri >> (PAIR_SH + 1)) & 1) << 6
            usel = ((ru >> PAIR_SH) & 1) == 1
            isel = ((ri >> PAIR_SH) & 1) == 1
            acc0 = bub[s] + bib[s]

            def h_step(h, acc):
                uw = plsc.load_gather(fub, [row, cu0 + h])
                vw = plsc.load_gather(fib, [row, ci0 + h])
                ulo, uhi = plsc.unpack(
                    plsc.bitcast(uw, jnp.bfloat16),
                    format=plsc.PackFormat.INTERLEAVED,
                    preferred_element_type=jnp.float32)
                vlo, vhi = plsc.unpack(
                    plsc.bitcast(vw, jnp.bfloat16),
                    format=plsc.PackFormat.INTERLEAVED,
                    preferred_element_type=jnp.float32)
                uu = jnp.where(usel, uhi, ulo)
                vv = jnp.where(isel, vhi, vlo)
                return acc + uu * vv

            acc = lax.fori_loop(0, HIDDEN, h_step, acc0)
            out_buf[pl.ds(c * CHUNK + g * LANES, LANES)] = acc

    pltpu.sync_copy(out_buf, out_hbm.at[pl.ds(wid * B_PER_W, B_PER_W)])


@jax.jit
def _run(users2d, items2d, fu_p, fi_p, bu_sum, bi_sum):
    mesh = plsc.VectorSubcoreMesh(
        core_axis_name="c", subcore_axis_name="s",
        num_cores=NUM_CORES, num_subcores=NUM_SUBCORES)
    return pl.kernel(
        _sc_body,
        out_type=jax.ShapeDtypeStruct((BATCH,), jnp.float32),
        mesh=mesh,
        compiler_params=pltpu.CompilerParams(
            needs_layout_passes=False, use_tc_tiling_on_sc=True),
        scratch_types=[
            pltpu.VMEM((NCHUNKS, CHUNK), jnp.int32),
            pltpu.VMEM((NCHUNKS, CHUNK), jnp.int32),
            pltpu.VMEM((NCHUNKS, CHUNK), jnp.int32),
            pltpu.VMEM((NCHUNKS, CHUNK), jnp.int32),
            pltpu.VMEM((CHUNK, 2 * HIDDEN), jnp.int32),
            pltpu.VMEM((CHUNK, 2 * HIDDEN), jnp.int32),
            pltpu.VMEM((CHUNK,), jnp.float32),
            pltpu.VMEM((CHUNK,), jnp.float32),
            pltpu.VMEM((B_PER_W,), jnp.float32),
            pltpu.SemaphoreType.DMA,
        ],
    )(users2d, items2d, fu_p, fi_p, bu_sum, bi_sum)


def kernel(users, items, user_factors, item_factors, user_biases,
           item_biases):
    grid = (NW * NCHUNKS, CHUNK)
    users2d = users.reshape(grid)
    items2d = items.reshape(grid)
    fu_p, fi_p, bu_sum, bi_sum = _prep(
        jnp.swapaxes(user_factors, 0, 1),
        jnp.swapaxes(item_factors, 0, 1),
        jnp.swapaxes(user_biases, 0, 1),
        jnp.swapaxes(item_biases, 0, 1))
    out = _run(users2d, items2d, fu_p, fi_p, bu_sum, bi_sum)
    return out.reshape(BATCH, 1)
